# baseline (speedup 1.0000x reference)
import functools

import jax
import jax.numpy as jnp
from jax.experimental import pallas as pl
from jax.experimental.pallas import tpu as pltpu

N = 10000
NFEAT = 256
NHID = 128
HB = 200
BR = 2 * HB
GRID = N // BR
PB = BR // 2


def _half_score(h_ref_block, support, benc, wdecT, hb):
    h = jnp.dot(h_ref_block.astype(jnp.bfloat16), support,
                preferred_element_type=jnp.float32)
    h = jnp.maximum(h + benc, 0.0)
    sv = jnp.dot(h, wdecT, preferred_element_type=jnp.float32)
    parity = jax.lax.broadcasted_iota(jnp.int32, (hb, 1), 0) % 2
    return jnp.where(parity == 0, sv[:, 0:1], sv[:, 1:2])


def _gcn_kernel(x_ref, adja_ref, adjb_ref, label_ref, wenc_ref, benc_ref,
                wdecT_ref, bdec_ref, out_ref, support_ref, acc_ref):
    i = pl.program_id(0)

    @pl.when(i == 0)
    def _init():
        support_ref[...] = jnp.dot(
            x_ref[...], wenc_ref[...],
            preferred_element_type=jnp.float32).astype(jnp.bfloat16)
        acc_ref[...] = jnp.zeros_like(acc_ref)

    support = support_ref[...]
    benc = benc_ref[...]
    wdecT = wdecT_ref[...]
    ta = _half_score(adja_ref[...], support, benc, wdecT, HB)
    tb = _half_score(adjb_ref[...], support, benc, wdecT, HB)
    t = jnp.concatenate([ta, tb], axis=0)

    prow = jax.lax.broadcasted_iota(jnp.int32, (PB, BR), 0)
    pcol = jax.lax.broadcasted_iota(jnp.int32, (PB, BR), 1)
    pair = (pcol // 2 == prow).astype(jnp.float32)
    logits = jnp.dot(pair, t, preferred_element_type=jnp.float32)
    logits = logits + bdec_ref[...]

    y = label_ref[pl.ds(i * PB, PB), :]
    terms = (jnp.maximum(logits, 0.0) - logits * y
             + jnp.log(1.0 + jnp.exp(-jnp.abs(logits))))
    acc_ref[...] += jnp.sum(terms)

    @pl.when(i == GRID - 1)
    def _fin():
        out_ref[...] = acc_ref[...] * (2.0 / N)


@functools.partial(jax.jit, static_argnames=("interpret",))
def kernel(x, adj, label, W_enc, b_enc, W_dec, b_dec, interpret=False):
    wdecT = W_dec[:, 0].reshape(2, NHID).T
    benc2 = b_enc.reshape(1, NHID)
    bdec2 = b_dec.reshape(1, 1)
    ylab = label[:, 0].reshape(2, GRID, PB // 2)
    ylab = ylab.transpose(1, 0, 2).reshape(N // 2, 1)

    out = pl.pallas_call(
        _gcn_kernel,
        grid=(GRID,),
        in_specs=[
            pl.BlockSpec((N, NFEAT), lambda i: (0, 0)),
            pl.BlockSpec((HB, N), lambda i: (i, 0)),
            pl.BlockSpec((HB, N), lambda i: (i + GRID, 0)),
            pl.BlockSpec((N // 2, 1), lambda i: (0, 0)),
            pl.BlockSpec((NFEAT, NHID), lambda i: (0, 0)),
            pl.BlockSpec((1, NHID), lambda i: (0, 0)),
            pl.BlockSpec((NHID, 2), lambda i: (0, 0)),
            pl.BlockSpec((1, 1), lambda i: (0, 0)),
        ],
        out_specs=pl.BlockSpec((1, 1), lambda i: (0, 0)),
        out_shape=jax.ShapeDtypeStruct((1, 1), jnp.float32),
        scratch_shapes=[
            pltpu.VMEM((N, NHID), jnp.bfloat16),
            pltpu.VMEM((1, 1), jnp.float32),
        ],
        interpret=interpret,
    )(x, adj, adj, ylab, W_enc, benc2, wdecT, bdec2)
    return out[0, 0]


# speedup vs baseline: 1.0516x; 1.0516x over previous
import functools

import jax
import jax.numpy as jnp
from jax.experimental import pallas as pl
from jax.experimental.pallas import tpu as pltpu

N = 10000
NFEAT = 256
NHID = 128
BR = 400
GRID = N // BR
PB = BR // 2


def _gcn_kernel(x_ref, adj_ref, label_ref, wenc_ref, benc_ref, wdec_ref,
                bdec_ref, out_ref, support_ref, acc_ref):
    i = pl.program_id(0)

    @pl.when(i == 0)
    def _init():
        support_ref[...] = jnp.dot(
            x_ref[...], wenc_ref[...],
            preferred_element_type=jnp.float32).astype(jnp.bfloat16)
        acc_ref[...] = jnp.zeros_like(acc_ref)

    h = jnp.dot(adj_ref[...].astype(jnp.bfloat16), support_ref[...],
                preferred_element_type=jnp.float32)
    h = jnp.maximum(h + benc_ref[...], 0.0)

    parity = jax.lax.broadcasted_iota(jnp.int32, (BR, 1), 0) % 2
    w_sel = jnp.where(parity == 0, wdec_ref[0:1, :], wdec_ref[1:2, :])
    s = jnp.sum(h * w_sel, axis=1, keepdims=True)

    prow = jax.lax.broadcasted_iota(jnp.int32, (PB, BR), 0)
    pcol = jax.lax.broadcasted_iota(jnp.int32, (PB, BR), 1)
    pair = (pcol // 2 == prow).astype(jnp.float32)
    logits = jnp.dot(pair, s, preferred_element_type=jnp.float32)
    logits = logits + bdec_ref[...]

    y = label_ref[pl.ds(i * PB, PB), :]
    terms = (jnp.maximum(logits, 0.0) - logits * y
             + jnp.log(1.0 + jnp.exp(-jnp.abs(logits))))
    acc_ref[...] += jnp.sum(terms)

    @pl.when(i == GRID - 1)
    def _fin():
        out_ref[...] = acc_ref[...] * (2.0 / N)


@functools.partial(jax.jit, static_argnames=("interpret",))
def kernel(x, adj, label, W_enc, b_enc, W_dec, b_dec, interpret=False):
    wdec2 = W_dec[:, 0].reshape(2, NHID)
    benc2 = b_enc.reshape(1, NHID)
    bdec2 = b_dec.reshape(1, 1)

    out = pl.pallas_call(
        _gcn_kernel,
        grid=(GRID,),
        in_specs=[
            pl.BlockSpec((N, NFEAT), lambda i: (0, 0)),
            pl.BlockSpec((BR, N), lambda i: (i, 0)),
            pl.BlockSpec((N // 2, 1), lambda i: (0, 0)),
            pl.BlockSpec((NFEAT, NHID), lambda i: (0, 0)),
            pl.BlockSpec((1, NHID), lambda i: (0, 0)),
            pl.BlockSpec((2, NHID), lambda i: (0, 0)),
            pl.BlockSpec((1, 1), lambda i: (0, 0)),
        ],
        out_specs=pl.BlockSpec((1, 1), lambda i: (0, 0)),
        out_shape=jax.ShapeDtypeStruct((1, 1), jnp.float32),
        scratch_shapes=[
            pltpu.VMEM((N, NHID), jnp.bfloat16),
            pltpu.VMEM((1, 1), jnp.float32),
        ],
        interpret=interpret,
    )(x, adj, label, W_enc, benc2, wdec2, bdec2)
    return out[0, 0]
